# baseline (speedup 1.0000x reference)
import functools

import jax
import jax.numpy as jnp
from jax import lax
from jax.experimental import pallas as pl
from jax.experimental.pallas import tpu as pltpu
from jax.experimental.pallas import tpu_sc as plsc

_N, _D, _B, _S, _T = 16384, 256, 16, 512, 32768
_NC, _NS = 2, 16
_NW = _NC * _NS
_TPW = _T // _NW
_CH = 64
_NCH = _TPW // _CH
_K = 4
_L = 16


def _sc_body(x_hbm, nb_hbm, sm_hbm, sb_hbm, bl_hbm,
             xs_hbm, nm_hbm,
             nb_v, bl_v, sm_v, sb_v, idx_v, cum_v,
             bufs, gsems, osems, nmsem):
    cid = lax.axis_index("c")
    sid = lax.axis_index("s")
    wid = sid * _NC + cid
    base = wid * _TPW

    pltpu.sync_copy(nb_hbm, nb_v)
    pltpu.sync_copy(bl_hbm, bl_v)
    pltpu.sync_copy(sm_hbm.at[pl.ds(base, _TPW)], sm_v)
    pltpu.sync_copy(sb_hbm.at[pl.ds(base, _TPW)], sb_v)

    bvec = lax.iota(jnp.int32, _L)
    lo = jnp.zeros((_L,), jnp.int32)
    hi = jnp.full((_L,), _N, jnp.int32)
    for _ in range(15):
        active = lo < hi
        mid = (lo + hi) >> 1
        midc = jnp.minimum(mid, _N - 1)
        v = plsc.load_gather(nb_v, [midc])
        less = v < bvec
        lo = jnp.where(jnp.logical_and(active, less), mid + 1, lo)
        hi = jnp.where(jnp.logical_and(active, jnp.logical_not(less)), mid, hi)
    cum_v[...] = lo

    for j in range(_TPW // _L):
        off = j * _L
        sbv = sb_v[pl.ds(off, _L)]
        g = plsc.load_gather(bl_v, [sbv])
        c = plsc.load_gather(cum_v, [g])
        nm = jnp.minimum(sm_v[pl.ds(off, _L)] + c, _N - 1)
        idx_v[j // (_CH // _L), pl.ds((j % (_CH // _L)) * _L, _L)] = nm
    nm_op = pltpu.make_async_copy(idx_v, nm_hbm.at[wid], nmsem)
    nm_op.start()

    gops = [None] * _NCH
    oops = [None] * _NCH
    for i in range(_K - 1):
        gops[i] = pltpu.make_async_copy(
            x_hbm.at[idx_v.at[i]], bufs.at[i % _K], gsems.at[i % _K])
        gops[i].start()
    for cb in range(_NCH):
        gops[cb].wait()
        oops[cb] = pltpu.make_async_copy(
            bufs.at[cb % _K], xs_hbm.at[pl.ds(base + cb * _CH, _CH)],
            osems.at[cb % _K])
        oops[cb].start()
        nxt = cb + _K - 1
        if nxt < _NCH:
            if nxt - _K >= 0:
                oops[nxt - _K].wait()
            gops[nxt] = pltpu.make_async_copy(
                x_hbm.at[idx_v.at[nxt]], bufs.at[nxt % _K], gsems.at[nxt % _K])
            gops[nxt].start()
    for cb in range(max(0, _NCH - _K), _NCH):
        if oops[cb] is not None:
            oops[cb].wait()
    nm_op.wait()


_sc_parse = functools.partial(
    pl.kernel,
    out_type=(
        jax.ShapeDtypeStruct((_T, _D), jnp.float32),
        jax.ShapeDtypeStruct((_NW, _NCH, _CH), jnp.int32),
    ),
    mesh=plsc.VectorSubcoreMesh(core_axis_name="c", subcore_axis_name="s"),
    compiler_params=pltpu.CompilerParams(needs_layout_passes=False),
    scratch_types=[
        pltpu.VMEM((_N,), jnp.int32),
        pltpu.VMEM((_S,), jnp.int32),
        pltpu.VMEM((_TPW,), jnp.int32),
        pltpu.VMEM((_TPW,), jnp.int32),
        pltpu.VMEM((_NCH, _CH), jnp.int32),
        pltpu.VMEM((_L,), jnp.int32),
        pltpu.VMEM((_K, _CH, _D), jnp.float32),
        pltpu.SemaphoreType.DMA((_K,)),
        pltpu.SemaphoreType.DMA((_K,)),
        pltpu.SemaphoreType.DMA,
    ],
)(_sc_body)


@jax.jit
def kernel(x, node_batch, sub_mapping, sub_batch, belong):
    nb = node_batch.astype(jnp.int32)
    sm = sub_mapping.astype(jnp.int32)
    sb = sub_batch.astype(jnp.int32)
    bl = belong.astype(jnp.int32)
    x_sub, nm = _sc_parse(x, nb, sm, sb, bl)
    return x_sub, nm.reshape(_T), bl


# speedup vs baseline: 1.0292x; 1.0292x over previous
import functools

import jax
import jax.numpy as jnp
from jax import lax
from jax.experimental import pallas as pl
from jax.experimental.pallas import tpu as pltpu
from jax.experimental.pallas import tpu_sc as plsc

_N, _D, _B, _S, _T = 16384, 256, 16, 512, 32768
_NC, _NS = 2, 16
_NW = _NC * _NS
_TPW = _T // _NW
_CH = 128
_NCH = _TPW // _CH
_K = 3
_L = 16


def _sc_body(x_hbm, nb_hbm, sm_hbm, sb_hbm, bl_hbm,
             xs_hbm, nm_hbm,
             nb_v, bl_v, sm_v, sb_v, idx_v, cum_v,
             bufs, gsems, osems, nmsem):
    cid = lax.axis_index("c")
    sid = lax.axis_index("s")
    wid = sid * _NC + cid
    base = wid * _TPW

    pltpu.sync_copy(nb_hbm, nb_v)
    pltpu.sync_copy(bl_hbm, bl_v)
    pltpu.sync_copy(sm_hbm.at[pl.ds(base, _TPW)], sm_v)
    pltpu.sync_copy(sb_hbm.at[pl.ds(base, _TPW)], sb_v)

    bvec = lax.iota(jnp.int32, _L)
    lo = jnp.zeros((_L,), jnp.int32)
    hi = jnp.full((_L,), _N, jnp.int32)
    for _ in range(15):
        active = lo < hi
        mid = (lo + hi) >> 1
        midc = jnp.minimum(mid, _N - 1)
        v = plsc.load_gather(nb_v, [midc])
        less = v < bvec
        lo = jnp.where(jnp.logical_and(active, less), mid + 1, lo)
        hi = jnp.where(jnp.logical_and(active, jnp.logical_not(less)), mid, hi)
    cum_v[...] = lo

    for j in range(_TPW // _L):
        off = j * _L
        sbv = sb_v[pl.ds(off, _L)]
        g = plsc.load_gather(bl_v, [sbv])
        c = plsc.load_gather(cum_v, [g])
        nm = jnp.minimum(sm_v[pl.ds(off, _L)] + c, _N - 1)
        idx_v[j // (_CH // _L), pl.ds((j % (_CH // _L)) * _L, _L)] = nm
    nm_op = pltpu.make_async_copy(idx_v, nm_hbm.at[wid], nmsem)
    nm_op.start()

    gops = [None] * _NCH
    oops = [None] * _NCH
    for i in range(_K - 1):
        gops[i] = pltpu.make_async_copy(
            x_hbm.at[idx_v.at[i]], bufs.at[i % _K], gsems.at[i % _K])
        gops[i].start()
    for cb in range(_NCH):
        gops[cb].wait()
        oops[cb] = pltpu.make_async_copy(
            bufs.at[cb % _K], xs_hbm.at[pl.ds(base + cb * _CH, _CH)],
            osems.at[cb % _K])
        oops[cb].start()
        nxt = cb + _K - 1
        if nxt < _NCH:
            if nxt - _K >= 0:
                oops[nxt - _K].wait()
            gops[nxt] = pltpu.make_async_copy(
                x_hbm.at[idx_v.at[nxt]], bufs.at[nxt % _K], gsems.at[nxt % _K])
            gops[nxt].start()
    for cb in range(max(0, _NCH - _K), _NCH):
        if oops[cb] is not None:
            oops[cb].wait()
    nm_op.wait()


_sc_parse = functools.partial(
    pl.kernel,
    out_type=(
        jax.ShapeDtypeStruct((_T, _D), jnp.float32),
        jax.ShapeDtypeStruct((_NW, _NCH, _CH), jnp.int32),
    ),
    mesh=plsc.VectorSubcoreMesh(core_axis_name="c", subcore_axis_name="s"),
    compiler_params=pltpu.CompilerParams(needs_layout_passes=False),
    scratch_types=[
        pltpu.VMEM((_N,), jnp.int32),
        pltpu.VMEM((_S,), jnp.int32),
        pltpu.VMEM((_TPW,), jnp.int32),
        pltpu.VMEM((_TPW,), jnp.int32),
        pltpu.VMEM((_NCH, _CH), jnp.int32),
        pltpu.VMEM((_L,), jnp.int32),
        pltpu.VMEM((_K, _CH, _D), jnp.float32),
        pltpu.SemaphoreType.DMA((_K,)),
        pltpu.SemaphoreType.DMA((_K,)),
        pltpu.SemaphoreType.DMA,
    ],
)(_sc_body)


@jax.jit
def kernel(x, node_batch, sub_mapping, sub_batch, belong):
    nb = node_batch.astype(jnp.int32)
    sm = sub_mapping.astype(jnp.int32)
    sb = sub_batch.astype(jnp.int32)
    bl = belong.astype(jnp.int32)
    x_sub, nm = _sc_parse(x, nb, sm, sb, bl)
    return x_sub, nm.reshape(_T), bl


# speedup vs baseline: 1.0997x; 1.0684x over previous
import functools

import jax
import jax.numpy as jnp
from jax import lax
from jax.experimental import pallas as pl
from jax.experimental.pallas import tpu as pltpu
from jax.experimental.pallas import tpu_sc as plsc

_N, _D, _B, _S, _T = 16384, 256, 16, 512, 32768
_NC, _NS = 2, 16
_NW = _NC * _NS
_TPW = _T // _NW
_CH = 128
_NCH = _TPW // _CH
_K = 3
_L = 16


def _sc_body(x_hbm, nb_hbm, sm_hbm, sb_hbm, bl_hbm,
             xs_hbm, nm_hbm,
             nb_v, bl_v, sm_v, sb_v, idx_v, cum_v,
             bufs, gsems, osems, nmsem):
    cid = lax.axis_index("c")
    sid = lax.axis_index("s")
    wid = sid * _NC + cid
    base = wid * _TPW

    nb_cp = pltpu.make_async_copy(nb_hbm, nb_v, gsems.at[0])
    nb_cp.start()
    bl_cp = pltpu.make_async_copy(bl_hbm, bl_v, gsems.at[1])
    bl_cp.start()
    sm_cp = pltpu.make_async_copy(sm_hbm.at[pl.ds(base, _TPW)], sm_v,
                                  gsems.at[2])
    sm_cp.start()
    sb_cp = pltpu.make_async_copy(sb_hbm.at[pl.ds(base, _TPW)], sb_v, osems.at[0])
    sb_cp.start()
    nb_cp.wait()

    bvec = lax.iota(jnp.int32, _L)
    lo = jnp.zeros((_L,), jnp.int32)
    hi = jnp.full((_L,), _N, jnp.int32)
    for _ in range(15):
        active = lo < hi
        mid = (lo + hi) >> 1
        midc = jnp.minimum(mid, _N - 1)
        v = plsc.load_gather(nb_v, [midc])
        less = v < bvec
        lo = jnp.where(jnp.logical_and(active, less), mid + 1, lo)
        hi = jnp.where(jnp.logical_and(active, jnp.logical_not(less)), mid, hi)
    cum_v[...] = lo
    bl_cp.wait()
    sm_cp.wait()
    sb_cp.wait()

    def _fill_row(cb):
        for jj in range(_CH // _L):
            off = cb * _CH + jj * _L
            sbv = sb_v[pl.ds(off, _L)]
            g = plsc.load_gather(bl_v, [sbv])
            c = plsc.load_gather(cum_v, [g])
            nm = jnp.minimum(sm_v[pl.ds(off, _L)] + c, _N - 1)
            idx_v[cb, pl.ds(jj * _L, _L)] = nm

    gops = [None] * _NCH
    oops = [None] * _NCH
    for i in range(_K - 1):
        _fill_row(i)
        gops[i] = pltpu.make_async_copy(
            x_hbm.at[idx_v.at[i]], bufs.at[i % _K], gsems.at[i % _K])
        gops[i].start()
    for cb in range(_NCH):
        nxt = cb + _K - 1
        if nxt < _NCH:
            _fill_row(nxt)
            if nxt - _K >= 0:
                oops[nxt - _K].wait()
            gops[nxt] = pltpu.make_async_copy(
                x_hbm.at[idx_v.at[nxt]], bufs.at[nxt % _K], gsems.at[nxt % _K])
            gops[nxt].start()
        gops[cb].wait()
        oops[cb] = pltpu.make_async_copy(
            bufs.at[cb % _K], xs_hbm.at[pl.ds(base + cb * _CH, _CH)],
            osems.at[cb % _K])
        oops[cb].start()
    nm_op = pltpu.make_async_copy(idx_v, nm_hbm.at[wid], nmsem)
    nm_op.start()
    for cb in range(max(0, _NCH - _K), _NCH):
        if oops[cb] is not None:
            oops[cb].wait()
    nm_op.wait()


_sc_parse = functools.partial(
    pl.kernel,
    out_type=(
        jax.ShapeDtypeStruct((_T, _D), jnp.float32),
        jax.ShapeDtypeStruct((_NW, _NCH, _CH), jnp.int32),
    ),
    mesh=plsc.VectorSubcoreMesh(core_axis_name="c", subcore_axis_name="s"),
    compiler_params=pltpu.CompilerParams(needs_layout_passes=False),
    scratch_types=[
        pltpu.VMEM((_N,), jnp.int32),
        pltpu.VMEM((_S,), jnp.int32),
        pltpu.VMEM((_TPW,), jnp.int32),
        pltpu.VMEM((_TPW,), jnp.int32),
        pltpu.VMEM((_NCH, _CH), jnp.int32),
        pltpu.VMEM((_L,), jnp.int32),
        pltpu.VMEM((_K, _CH, _D), jnp.float32),
        pltpu.SemaphoreType.DMA((_K,)),
        pltpu.SemaphoreType.DMA((_K,)),
        pltpu.SemaphoreType.DMA,
    ],
)(_sc_body)


@jax.jit
def kernel(x, node_batch, sub_mapping, sub_batch, belong):
    nb = node_batch.astype(jnp.int32)
    sm = sub_mapping.astype(jnp.int32)
    sb = sub_batch.astype(jnp.int32)
    bl = belong.astype(jnp.int32)
    x_sub, nm = _sc_parse(x, nb, sm, sb, bl)
    return x_sub, nm.reshape(_T), bl
